# baseline (speedup 1.0000x reference)
import functools

import jax
import jax.numpy as jnp
from jax import lax
from jax.experimental import pallas as pl
from jax.experimental.pallas import tpu as pltpu
from jax.experimental.pallas import tpu_sc as plsc

F = 26
V = 100000
D = 32
B = 16384
LOG2_B = 14
PACK = 128 // D

NC = 2
NS = 16
NW = NC * NS
TOTAL_ROWS = F * B
RPW = TOTAL_ROWS // NW
CHUNK = 416
NCHUNK = RPW // CHUNK


def _sc_gather_body(tables_hbm, idx_hbm, out_hbm,
                    idx_a, idx_b, wide_a, wide_b, sem_a, sem_b):
    wid = lax.axis_index("s") * NC + lax.axis_index("c")
    base = pl.multiple_of(wid * RPW, CHUNK)
    idx_bufs = (idx_a, idx_b)
    wide_bufs = (wide_a, wide_b)
    sems = (sem_a, sem_b)

    def stage_indices(c, k):
        pltpu.sync_copy(idx_hbm.at[pl.ds(base + c * CHUNK, CHUNK)], idx_bufs[k])

        def fix(i, _):
            off = pl.multiple_of(i * 16, 16)
            pos = base + c * CHUNK + off + lax.iota(jnp.int32, 16)
            fld = lax.shift_right_logical(pos, LOG2_B)
            idx_bufs[k][pl.ds(off, 16)] = idx_bufs[k][pl.ds(off, 16)] + fld * V
            return 0

        lax.fori_loop(0, CHUNK // 16, fix, 0)

    def drain(c):
        k = c % 2
        pltpu.sync_copy(wide_bufs[k].at[:, pl.ds(0, D)],
                        out_hbm.at[pl.ds(base + c * CHUNK, CHUNK)])

    prev = None
    for c in range(NCHUNK):
        k = c % 2
        stage_indices(c, k)
        cp = pltpu.async_copy(tables_hbm.at[idx_bufs[k]], wide_bufs[k], sems[k])
        if prev is not None:
            prev.wait()
            drain(c - 1)
        prev = cp
    prev.wait()
    drain(NCHUNK - 1)


@functools.lru_cache(maxsize=None)
def _sc_gather():
    return pl.kernel(
        _sc_gather_body,
        mesh=plsc.VectorSubcoreMesh(core_axis_name="c", subcore_axis_name="s"),
        out_type=jax.ShapeDtypeStruct((TOTAL_ROWS, D), jnp.float32),
        scratch_types=[
            pltpu.VMEM((CHUNK,), jnp.int32),
            pltpu.VMEM((CHUNK,), jnp.int32),
            pltpu.VMEM((CHUNK, 128), jnp.float32),
            pltpu.VMEM((CHUNK, 128), jnp.float32),
            pltpu.SemaphoreType.DMA,
            pltpu.SemaphoreType.DMA,
        ],
        compiler_params=pltpu.CompilerParams(use_tc_tiling_on_sc=False),
    )


BT = 2048
BTP = BT // PACK


def _mlp_body(embs_ref, w1_ref, b1_ref, w2_ref, b2_ref, out_ref):
    acc = jnp.zeros((BTP, 128), jnp.float32)
    for f in range(F):
        acc = acc + jnp.dot(embs_ref[f], w1_ref[f],
                            preferred_element_type=jnp.float32)
    h = jnp.maximum(acc + b1_ref[...], 0.0)
    out_ref[...] = jnp.dot(h, w2_ref[...],
                           preferred_element_type=jnp.float32) + b2_ref[...]


def _mlp(embs_packed, W1p, b1p, W2p, b2p):
    return pl.pallas_call(
        _mlp_body,
        grid=(B // BT,),
        in_specs=[
            pl.BlockSpec((F, BTP, 128), lambda i: (0, i, 0)),
            pl.BlockSpec((F, 128, 128), lambda i: (0, 0, 0)),
            pl.BlockSpec((1, 128), lambda i: (0, 0)),
            pl.BlockSpec((128, 128), lambda i: (0, 0)),
            pl.BlockSpec((1, 128), lambda i: (0, 0)),
        ],
        out_specs=pl.BlockSpec((BTP, 128), lambda i: (i, 0)),
        out_shape=jax.ShapeDtypeStruct((B // PACK, 128), jnp.float32),
    )(embs_packed, W1p, b1p, W2p, b2p)


def kernel(indices, tables, W1, b1, W2, b2):
    tables_wide = jnp.pad(tables, ((0, 0), (0, 0), (0, 128 - D))) \
        .reshape(F * V, 128)
    idx_flat = indices.reshape(TOTAL_ROWS)
    embs = _sc_gather()(tables_wide, idx_flat)
    embs_packed = embs.reshape(F, B // PACK, 128)

    eye = jnp.eye(PACK, dtype=jnp.float32)
    W1p = jax.vmap(lambda w: jnp.kron(eye, w))(W1.reshape(F, D, D))
    W2p = jnp.kron(eye, W2)
    b1p = jnp.tile(b1.reshape(-1), PACK).reshape(1, 128)
    b2p = jnp.tile(b2.reshape(-1), PACK).reshape(1, 128)

    out_packed = _mlp(embs_packed, W1p, b1p, W2p, b2p)
    return embs


# speedup vs baseline: 1.4936x; 1.4936x over previous
import functools

import jax
import jax.numpy as jnp
from jax import lax
from jax.experimental import pallas as pl
from jax.experimental.pallas import tpu as pltpu
from jax.experimental.pallas import tpu_sc as plsc

F = 26
V = 100000
D = 32
B = 16384
LOG2_B = 14
PACK = 128 // D

NC = 2
NS = 16
NW = NC * NS
TOTAL_ROWS = F * B
RPW = TOTAL_ROWS // NW
CHUNK = 416
NCHUNK = RPW // CHUNK


def _sc_gather_body(tables_hbm, idx_hbm, out_hbm,
                    idx_a, idx_b, wide_a, wide_b, sem_a, sem_b):
    wid = lax.axis_index("s") * NC + lax.axis_index("c")
    base = pl.multiple_of(wid * RPW, CHUNK)
    idx_bufs = (idx_a, idx_b)
    wide_bufs = (wide_a, wide_b)
    sems = (sem_a, sem_b)

    def stage_indices(c, k):
        pltpu.sync_copy(idx_hbm.at[pl.ds(base + c * CHUNK, CHUNK)], idx_bufs[k])

        def fix(i, _):
            off = pl.multiple_of(i * 16, 16)
            pos = base + c * CHUNK + off + lax.iota(jnp.int32, 16)
            fld = lax.shift_right_logical(pos, LOG2_B)
            idx_bufs[k][pl.ds(off, 16)] = idx_bufs[k][pl.ds(off, 16)] + fld * V
            return 0

        lax.fori_loop(0, CHUNK // 16, fix, 0)

    def drain(c):
        k = c % 2
        pltpu.sync_copy(wide_bufs[k].at[:, pl.ds(0, D)],
                        out_hbm.at[pl.ds(base + c * CHUNK, CHUNK)])

    prev = None
    for c in range(NCHUNK):
        k = c % 2
        stage_indices(c, k)
        cp = pltpu.async_copy(tables_hbm.at[idx_bufs[k]], wide_bufs[k], sems[k])
        if prev is not None:
            prev.wait()
            drain(c - 1)
        prev = cp
    prev.wait()
    drain(NCHUNK - 1)


@functools.lru_cache(maxsize=None)
def _sc_gather():
    return pl.kernel(
        _sc_gather_body,
        mesh=plsc.VectorSubcoreMesh(core_axis_name="c", subcore_axis_name="s"),
        out_type=jax.ShapeDtypeStruct((TOTAL_ROWS, D), jnp.float32),
        scratch_types=[
            pltpu.VMEM((CHUNK,), jnp.int32),
            pltpu.VMEM((CHUNK,), jnp.int32),
            pltpu.VMEM((CHUNK, 128), jnp.float32),
            pltpu.VMEM((CHUNK, 128), jnp.float32),
            pltpu.SemaphoreType.DMA,
            pltpu.SemaphoreType.DMA,
        ],
        compiler_params=pltpu.CompilerParams(use_tc_tiling_on_sc=False),
    )


TV = 4096


def _transpose_body(xt_ref, out_ref):
    out_ref[0, :, pl.ds(0, D)] = xt_ref[0].T


def _transpose_tables(tables_t):
    grid_v = (V + TV - 1) // TV
    return pl.pallas_call(
        _transpose_body,
        grid=(F, grid_v),
        in_specs=[pl.BlockSpec((1, D, TV), lambda f, i: (f, 0, i))],
        out_specs=pl.BlockSpec((1, TV, 128), lambda f, i: (f, i, 0)),
        out_shape=jax.ShapeDtypeStruct((F, V, 128), jnp.float32),
    )(tables_t)


BT = 2048
BTP = BT // PACK


def _mlp_body(embs_ref, w1_ref, b1_ref, w2_ref, b2_ref, out_ref):
    acc = jnp.zeros((BTP, 128), jnp.float32)
    for f in range(F):
        acc = acc + jnp.dot(embs_ref[f], w1_ref[f],
                            preferred_element_type=jnp.float32)
    h = jnp.maximum(acc + b1_ref[...], 0.0)
    out_ref[...] = jnp.dot(h, w2_ref[...],
                           preferred_element_type=jnp.float32) + b2_ref[...]


def _mlp(embs_packed, W1p, b1p, W2p, b2p):
    return pl.pallas_call(
        _mlp_body,
        grid=(B // BT,),
        in_specs=[
            pl.BlockSpec((F, BTP, 128), lambda i: (0, i, 0)),
            pl.BlockSpec((F, 128, 128), lambda i: (0, 0, 0)),
            pl.BlockSpec((1, 128), lambda i: (0, 0)),
            pl.BlockSpec((128, 128), lambda i: (0, 0)),
            pl.BlockSpec((1, 128), lambda i: (0, 0)),
        ],
        out_specs=pl.BlockSpec((BTP, 128), lambda i: (i, 0)),
        out_shape=jax.ShapeDtypeStruct((B // PACK, 128), jnp.float32),
    )(embs_packed, W1p, b1p, W2p, b2p)


def kernel(indices, tables, W1, b1, W2, b2):
    tables_t = tables.transpose(0, 2, 1)
    tables_wide = _transpose_tables(tables_t).reshape(F * V, 128)
    idx_flat = indices.reshape(TOTAL_ROWS)
    embs = _sc_gather()(tables_wide, idx_flat)
    embs_packed = embs.reshape(F, B // PACK, 128)

    eye = jnp.eye(PACK, dtype=jnp.float32)
    W1p = jax.vmap(lambda w: jnp.kron(eye, w))(W1.reshape(F, D, D))
    W2p = jnp.kron(eye, W2)
    b1p = jnp.tile(b1.reshape(-1), PACK).reshape(1, 128)
    b2p = jnp.tile(b2.reshape(-1), PACK).reshape(1, 128)

    out_packed = _mlp(embs_packed, W1p, b1p, W2p, b2p)
    return out_packed.reshape(B, D)
